# baseline (speedup 1.0000x reference)
import functools

import jax
import jax.numpy as jnp
from jax import lax
from jax.experimental import pallas as pl
from jax.experimental.pallas import tpu as pltpu
from jax.experimental.pallas import tpu_sc as plsc

EPS = 1e-5
B, T, V, D = 16, 2048, 8192, 256
NC, NS, L = 2, 16, 16
NW = NC * NS
DPW = D // NW
ROWS = 512


def _prep_body(emb_ref, usage_ref, out_ref):
    inv = 1.0 / jnp.maximum(usage_ref[...], EPS)
    out_ref[...] = jnp.transpose(emb_ref[...] * inv)


def _prep(emb, usage2d):
    return pl.pallas_call(
        _prep_body,
        grid=(V // ROWS,),
        in_specs=[
            pl.BlockSpec((ROWS, D), lambda i: (i, 0)),
            pl.BlockSpec((ROWS, 1), lambda i: (i, 0)),
        ],
        out_specs=pl.BlockSpec((D, ROWS), lambda i: (0, i)),
        out_shape=jax.ShapeDtypeStruct((D, V), jnp.float32),
    )(emb, usage2d)


def _sc_body(embT_hbm, codes_hbm, out_hbm, tab_v, codes_v, out_v):
    wid = lax.axis_index("s") * NC + lax.axis_index("c")
    d0 = wid * DPW
    pltpu.sync_copy(embT_hbm.at[pl.ds(d0 * V, DPW * V)], tab_v)

    for b in range(B):
        pltpu.sync_copy(codes_hbm.at[pl.ds(b * T, T)], codes_v)

        def body(tc, carry):
            idx = codes_v[pl.ds(tc * L, L)]
            for j in range(DPW):
                vals = plsc.load_gather(tab_v, [idx + jnp.int32(j * V)])
                out_v[pl.ds(j * T + tc * L, L)] = vals
            return carry

        lax.fori_loop(0, T // L, body, 0)
        pltpu.sync_copy(out_v, out_hbm.at[pl.ds((b * D + d0) * T, DPW * T)])


_sc_gather = functools.partial(
    pl.kernel,
    out_type=jax.ShapeDtypeStruct((B * D * T,), jnp.float32),
    mesh=plsc.VectorSubcoreMesh(core_axis_name="c", subcore_axis_name="s"),
    compiler_params=pltpu.CompilerParams(needs_layout_passes=False),
    scratch_types=[
        pltpu.VMEM((DPW * V,), jnp.float32),
        pltpu.VMEM((T,), jnp.int32),
        pltpu.VMEM((DPW * T,), jnp.float32),
    ],
)(_sc_body)


@jax.jit
def kernel(codes, embedding_sum, cluster_usage):
    embT = _prep(embedding_sum, cluster_usage.reshape(V, 1))
    flat = _sc_gather(embT.reshape(D * V), codes.reshape(B * T))
    return flat.reshape(B, D, T)


# speedup vs baseline: 1.6562x; 1.6562x over previous
import functools

import jax
import jax.numpy as jnp
from jax import lax
from jax.experimental import pallas as pl
from jax.experimental.pallas import tpu as pltpu
from jax.experimental.pallas import tpu_sc as plsc

EPS = 1e-5
B, T, V, D = 16, 2048, 8192, 256
NC, NS, L = 2, 16, 16
NW = NC * NS
DPW = D // NW
ROWS = 512


def _prep_body(emb_ref, usage_ref, out_ref):
    inv = 1.0 / jnp.maximum(usage_ref[...], EPS)
    out_ref[...] = jnp.transpose(emb_ref[...] * inv)


def _prep(emb, usage2d):
    return pl.pallas_call(
        _prep_body,
        grid=(V // ROWS,),
        in_specs=[
            pl.BlockSpec((ROWS, D), lambda i: (i, 0)),
            pl.BlockSpec((ROWS, 1), lambda i: (i, 0)),
        ],
        out_specs=pl.BlockSpec((D, ROWS), lambda i: (0, i)),
        out_shape=jax.ShapeDtypeStruct((D, V), jnp.float32),
    )(emb, usage2d)


def _sc_body(embT_hbm, codes_hbm, out_hbm, tab_v, codes_v, out_v,
             sem_t, sem_c, sem_o):
    wid = lax.axis_index("s") * NC + lax.axis_index("c")
    d0 = wid * DPW
    tab_cp = pltpu.async_copy(embT_hbm.at[pl.ds(d0 * V, DPW * V)], tab_v, sem_t)
    code_cp = pltpu.async_copy(codes_hbm.at[pl.ds(0, T)],
                               codes_v.at[pl.ds(0, T)], sem_c)
    tab_cp.wait()

    out_cp = [None, None]
    for b in range(B):
        sc = (b % 2) * T
        so = (b % 2) * DPW * T
        code_cp.wait()
        if b + 1 < B:
            code_cp = pltpu.async_copy(
                codes_hbm.at[pl.ds((b + 1) * T, T)],
                codes_v.at[pl.ds(((b + 1) % 2) * T, T)], sem_c)
        if out_cp[b % 2] is not None:
            out_cp[b % 2].wait()

        @plsc.parallel_loop(0, T, step=L, unroll=4)
        def body(i):
            idx = codes_v[pl.ds(sc + i, L)]
            for j in range(DPW):
                vals = plsc.load_gather(tab_v, [idx + jnp.int32(j * V)])
                out_v[pl.ds(so + j * T + i, L)] = vals

        out_cp[b % 2] = pltpu.async_copy(
            out_v.at[pl.ds(so, DPW * T)],
            out_hbm.at[pl.ds((b * D + d0) * T, DPW * T)], sem_o)

    out_cp[0].wait()
    out_cp[1].wait()


_sc_gather = functools.partial(
    pl.kernel,
    out_type=jax.ShapeDtypeStruct((B * D * T,), jnp.float32),
    mesh=plsc.VectorSubcoreMesh(core_axis_name="c", subcore_axis_name="s"),
    compiler_params=pltpu.CompilerParams(needs_layout_passes=False),
    scratch_types=[
        pltpu.VMEM((DPW * V,), jnp.float32),
        pltpu.VMEM((2 * T,), jnp.int32),
        pltpu.VMEM((2 * DPW * T,), jnp.float32),
        pltpu.SemaphoreType.DMA,
        pltpu.SemaphoreType.DMA,
        pltpu.SemaphoreType.DMA,
    ],
)(_sc_body)


@jax.jit
def kernel(codes, embedding_sum, cluster_usage):
    embT = _prep(embedding_sum, cluster_usage.reshape(V, 1))
    flat = _sc_gather(embT.reshape(D * V), codes.reshape(B * T))
    return flat.reshape(B, D, T)


# speedup vs baseline: 2.3672x; 1.4293x over previous
import functools

import jax
import jax.numpy as jnp
from jax import lax
from jax.experimental import pallas as pl
from jax.experimental.pallas import tpu as pltpu
from jax.experimental.pallas import tpu_sc as plsc

EPS = 1e-5
B, T, V, D = 16, 2048, 8192, 256
NC, NS, L = 2, 16, 16
NW = NC * NS
DPW = D // NW
ROWS = 512


def _prep_body(emb_ref, usage_ref, out_ref):
    inv = 1.0 / jnp.maximum(usage_ref[...], EPS)
    out_ref[...] = jnp.transpose(emb_ref[...] * inv)


def _prep(emb, usage2d):
    return pl.pallas_call(
        _prep_body,
        grid=(V // ROWS,),
        in_specs=[
            pl.BlockSpec((ROWS, D), lambda i: (i, 0)),
            pl.BlockSpec((ROWS, 1), lambda i: (i, 0)),
        ],
        out_specs=pl.BlockSpec((D, ROWS), lambda i: (0, i)),
        out_shape=jax.ShapeDtypeStruct((D, V), jnp.float32),
    )(emb, usage2d)


def _sc_body(embT_hbm, codes_hbm, out_hbm, tab_v, codes_v, out_v,
             sem_t, sem_c, sem_o):
    wid = lax.axis_index("s") * NC + lax.axis_index("c")
    d0 = wid * DPW
    tab_cp = pltpu.async_copy(embT_hbm.at[pl.ds(d0 * V, DPW * V)], tab_v, sem_t)
    code_cp = pltpu.async_copy(codes_hbm.at[0, 0, :], codes_v.at[pl.ds(0, T)],
                               sem_c)
    tab_cp.wait()

    out_cp = [None, None]
    for b in range(B):
        sc = (b % 2) * T
        slot = b % 2
        code_cp.wait()
        if b + 1 < B:
            code_cp = pltpu.async_copy(
                codes_hbm.at[b + 1, 0, :],
                codes_v.at[pl.ds(((b + 1) % 2) * T, T)], sem_c)
        if out_cp[slot] is not None:
            out_cp[slot].wait()

        @plsc.parallel_loop(0, T, step=L, unroll=4)
        def body(i):
            idx = codes_v[pl.ds(sc + i, L)]
            for j in range(DPW):
                vals = plsc.load_gather(tab_v, [idx + jnp.int32(j * V)])
                out_v[slot, j, pl.ds(i, L)] = vals

        out_cp[slot] = pltpu.async_copy(
            out_v.at[slot], out_hbm.at[b, pl.ds(d0, DPW), :], sem_o)

    out_cp[0].wait()
    out_cp[1].wait()


_sc_gather = functools.partial(
    pl.kernel,
    out_type=jax.ShapeDtypeStruct((B, D, T), jnp.float32),
    mesh=plsc.VectorSubcoreMesh(core_axis_name="c", subcore_axis_name="s"),
    compiler_params=pltpu.CompilerParams(needs_layout_passes=False),
    scratch_types=[
        pltpu.VMEM((DPW * V,), jnp.float32),
        pltpu.VMEM((2 * T,), jnp.int32),
        pltpu.VMEM((2, DPW, T), jnp.float32),
        pltpu.SemaphoreType.DMA,
        pltpu.SemaphoreType.DMA,
        pltpu.SemaphoreType.DMA,
    ],
)(_sc_body)


@jax.jit
def kernel(codes, embedding_sum, cluster_usage):
    embT = _prep(embedding_sum, cluster_usage.reshape(V, 1))
    return _sc_gather(embT.reshape(D * V), codes)
